# baseline (speedup 1.0000x reference)
import functools

import jax
import jax.numpy as jnp
from jax import lax
from jax.experimental import pallas as pl
from jax.experimental.pallas import tpu as pltpu
from jax.experimental.pallas import tpu_sc as plsc

ALPHA = 0.2

N = 10000
F = 128
H = 128
K = 10
E = 320000

L = 16
NC = 2
NS = 16
NW = NC * NS

EPG = 64
N_PAD = 10240
NPT = N_PAD // NS
HPT = 640
HLAST = N - (NS - 1) * HPT
E_PAD = 327680
GT = E_PAD // EPG
GPT = GT // NW
G_SC0 = 288
G_SC1 = (GT - NS * G_SC0) // NS
DGPT = GT // NS
CH = 16


def _rsqrt_newton(d):
    ib = lax.bitcast_convert_type(d, jnp.int32)
    y = lax.bitcast_convert_type(jnp.int32(0x5F3759DF) - (ib >> 1), jnp.float32)
    for _ in range(3):
        y = y * (1.5 - 0.5 * d * y * y)
    return y


def _sc_kernel_body(row2, col2, ew2, xw, part_out, dis_out,
                    nvec_sh, hacc_sh,
                    rowv, colv, eww, gbuf0, gbuf1, sbuf0, sbuf1,
                    degv, disl,
                    gsem0, gsem1, ssem0, ssem1, dsem):
    cid = lax.axis_index("c")
    sid = lax.axis_index("s")
    nbase = sid * NPT
    hbase = sid * HPT

    z = jnp.zeros((L,), jnp.float32)

    def zdeg(i, c):
        degv[pl.ds(i * L, L)] = z
        return c
    lax.fori_loop(0, NPT // L, zdeg, 0)
    pltpu.sync_copy(degv, nvec_sh.at[pl.ds(nbase, NPT)])

    def zrow(i, c):
        for v in range(H // L):
            sbuf0[i, pl.ds(v * L, L)] = z
        return c
    lax.fori_loop(0, EPG, zrow, 0)

    @pl.when(sid < NS - 1)
    def _():
        zs = [pltpu.async_copy(sbuf0, hacc_sh.at[pl.ds(hbase + r * EPG, EPG)],
                               dsem)
              for r in range(HPT // EPG)]
        for d in zs:
            d.wait()

    @pl.when(sid == NS - 1)
    def _():
        zs = [pltpu.async_copy(sbuf0, hacc_sh.at[pl.ds(hbase + r * EPG, EPG)],
                               dsem)
              for r in range(HLAST // EPG)]
        zs.append(pltpu.async_copy(
            sbuf0.at[pl.ds(0, HLAST % EPG)],
            hacc_sh.at[pl.ds(hbase + (HLAST // EPG) * EPG, HLAST % EPG)],
            dsem))
        for d in zs:
            d.wait()

    plsc.subcore_barrier()

    dg0 = sid * DGPT
    scope_deg = jax.named_scope("deg_phase")
    scope_deg.__enter__()

    def deg_chunk(ci, c):
        gb = dg0 + ci * CH
        pltpu.sync_copy(col2.at[pl.ds(gb, CH)], colv)
        pltpu.sync_copy(ew2.at[pl.ds(gb, CH)], eww)
        ds_ = [pltpu.async_copy(eww.at[g], nvec_sh.at[colv.at[g]], dsem,
                                add=True)
               for g in range(CH)]
        for d in ds_:
            d.wait()
        return c
    lax.fori_loop(0, DGPT // CH, deg_chunk, 0)

    plsc.subcore_barrier()
    scope_deg.__exit__(None, None, None)

    pltpu.sync_copy(nvec_sh.at[pl.ds(nbase, NPT)], degv)

    def dis_step(i, c):
        d = degv[pl.ds(i * L, L)] + 1.0
        degv[pl.ds(i * L, L)] = _rsqrt_newton(d)
        return c
    lax.fori_loop(0, NPT // L, dis_step, 0)

    pltpu.sync_copy(degv, nvec_sh.at[pl.ds(nbase, NPT)])

    @pl.when(cid == 0)
    def _():
        pltpu.sync_copy(degv, dis_out.at[pl.ds(nbase, NPT)])

    plsc.subcore_barrier()

    pltpu.sync_copy(nvec_sh, disl)
    scope_msg = jax.named_scope("msg_phase")
    scope_msg.__enter__()

    g0 = jnp.where(cid == 0, sid * G_SC0, NS * G_SC0 + sid * G_SC1)
    nchunk = jnp.where(cid == 0, G_SC0 // CH, G_SC1 // CH)

    def process(g, i, gbuf, gsem, sbuf, ssem):
        pltpu.make_async_copy(xw.at[rowv.at[g]], gbuf, gsem).wait()

        @pl.when(i > 0)
        def _():
            pltpu.make_async_copy(sbuf, hacc_sh.at[colv.at[g]], ssem).wait()

        def scale(j8, cc):
            ri = rowv[g, pl.ds(j8 * L, L)]
            ev = eww[g, pl.ds(j8 * L, L)]
            nv = plsc.load_gather(disl, [ri]) * ev
            for jj in range(L):
                s = nv[jj]
                j = j8 * L + jj
                for v in range(H // L):
                    sbuf[j, pl.ds(v * L, L)] = gbuf[j, pl.ds(v * L, L)] * s
            return cc
        lax.fori_loop(0, EPG // L, scale, 0)

        pltpu.async_copy(sbuf, hacc_sh.at[colv.at[g]], ssem, add=True)
        gg = g + 2

        @pl.when(gg < CH)
        def _():
            pltpu.async_copy(xw.at[rowv.at[gg]], gbuf, gsem)

    def msg_chunk(ci, c):
        gb = g0 + ci * CH
        pltpu.sync_copy(row2.at[pl.ds(gb, CH)], rowv)
        pltpu.sync_copy(col2.at[pl.ds(gb, CH)], colv)
        pltpu.sync_copy(ew2.at[pl.ds(gb, CH)], eww)
        pltpu.async_copy(xw.at[rowv.at[0]], gbuf0, gsem0)
        pltpu.async_copy(xw.at[rowv.at[1]], gbuf1, gsem1)

        def pair(i, cc):
            process(2 * i, i, gbuf0, gsem0, sbuf0, ssem0)
            process(2 * i + 1, i, gbuf1, gsem1, sbuf1, ssem1)
            return cc
        lax.fori_loop(0, CH // 2, pair, 0)

        pltpu.make_async_copy(sbuf0, hacc_sh.at[colv.at[0]], ssem0).wait()
        pltpu.make_async_copy(sbuf1, hacc_sh.at[colv.at[1]], ssem1).wait()
        return c
    lax.fori_loop(0, nchunk, msg_chunk, 0)

    plsc.subcore_barrier()
    scope_msg.__exit__(None, None, None)

    @pl.when(sid < NS - 1)
    def _():
        pltpu.sync_copy(hacc_sh.at[pl.ds(hbase, HPT)],
                        part_out.at[cid, pl.ds(hbase, HPT)])

    @pl.when(sid == NS - 1)
    def _():
        pltpu.sync_copy(hacc_sh.at[pl.ds(hbase, HLAST)],
                        part_out.at[cid, pl.ds(hbase, HLAST)])


def _make_sc_kernel():
    mesh = plsc.VectorSubcoreMesh(core_axis_name="c", subcore_axis_name="s")
    return functools.partial(
        pl.kernel,
        mesh=mesh,
        compiler_params=pltpu.CompilerParams(needs_layout_passes=False),
        out_type=[
            jax.ShapeDtypeStruct((NC, N, H), jnp.float32),
            jax.ShapeDtypeStruct((N_PAD,), jnp.float32),
        ],
        scratch_types=[
            pltpu.VMEM_SHARED((N_PAD,), jnp.float32),
            pltpu.VMEM_SHARED((N, H), jnp.float32),
            pltpu.VMEM((CH, EPG), jnp.int32),
            pltpu.VMEM((CH, EPG), jnp.int32),
            pltpu.VMEM((CH, EPG), jnp.float32),
            pltpu.VMEM((EPG, H), jnp.float32),
            pltpu.VMEM((EPG, H), jnp.float32),
            pltpu.VMEM((EPG, H), jnp.float32),
            pltpu.VMEM((EPG, H), jnp.float32),
            pltpu.VMEM((NPT,), jnp.float32),
            pltpu.VMEM((N_PAD,), jnp.float32),
            pltpu.SemaphoreType.DMA,
            pltpu.SemaphoreType.DMA,
            pltpu.SemaphoreType.DMA,
            pltpu.SemaphoreType.DMA,
            pltpu.SemaphoreType.DMA,
        ],
    )(_sc_kernel_body)


def _mm_body(x_ref, w_ref, o_ref):
    o_ref[...] = jnp.dot(x_ref[...], w_ref[...],
                         preferred_element_type=jnp.float32)


def _fin_body(p0, p1, xwr, disr, br, mur, hout, qout):
    d = disr[...]
    h = d * (p0[0] + p1[0]) + (d * d) * xwr[...] + br[...]
    hout[...] = h
    cols = []
    for k in range(K):
        mk = mur[k, :][None, :]
        dd = h - mk
        cols.append(jnp.sum(dd * dd, axis=1, keepdims=True))
    d2 = jnp.concatenate(cols, axis=1)
    u = 1.0 / (1.0 + d2 / ALPHA + 1e-8)
    qq = jnp.exp((ALPHA + 1.0) * jnp.log(u))
    qout[...] = qq / jnp.sum(qq, axis=1, keepdims=True)


def kernel(x, edge_index, edge_attr, W, b, mu):
    row = edge_index[0]
    col = edge_index[1]
    pad = E_PAD - E
    row2 = jnp.concatenate([row, jnp.zeros((pad,), jnp.int32)]).reshape(GT, EPG)
    col2 = jnp.concatenate([col, jnp.zeros((pad,), jnp.int32)]).reshape(GT, EPG)
    ew2 = jnp.concatenate([edge_attr,
                           jnp.zeros((pad,), jnp.float32)]).reshape(GT, EPG)

    BLK = 1000
    xw = pl.pallas_call(
        _mm_body,
        grid=(N // BLK,),
        in_specs=[pl.BlockSpec((BLK, F), lambda i: (i, 0)),
                  pl.BlockSpec((F, H), lambda i: (0, 0))],
        out_specs=pl.BlockSpec((BLK, H), lambda i: (i, 0)),
        out_shape=jax.ShapeDtypeStruct((N, H), jnp.float32),
    )(x, W)

    parts, dis = _make_sc_kernel()(row2, col2, ew2, xw)

    dis2 = dis[:N].reshape(N, 1)
    b2 = b.reshape(1, H)

    h, q = pl.pallas_call(
        _fin_body,
        grid=(N // BLK,),
        in_specs=[pl.BlockSpec((1, BLK, H), lambda i: (0, i, 0)),
                  pl.BlockSpec((1, BLK, H), lambda i: (1, i, 0)),
                  pl.BlockSpec((BLK, H), lambda i: (i, 0)),
                  pl.BlockSpec((BLK, 1), lambda i: (i, 0)),
                  pl.BlockSpec((1, H), lambda i: (0, 0)),
                  pl.BlockSpec((K, H), lambda i: (0, 0))],
        out_specs=[pl.BlockSpec((BLK, H), lambda i: (i, 0)),
                   pl.BlockSpec((BLK, K), lambda i: (i, 0))],
        out_shape=[jax.ShapeDtypeStruct((N, H), jnp.float32),
                   jax.ShapeDtypeStruct((N, K), jnp.float32)],
    )(parts, parts, xw, dis2, b2, mu)

    return (h, q)


# speedup vs baseline: 1.0131x; 1.0131x over previous
import functools

import jax
import jax.numpy as jnp
from jax import lax
from jax.experimental import pallas as pl
from jax.experimental.pallas import tpu as pltpu
from jax.experimental.pallas import tpu_sc as plsc

ALPHA = 0.2

N = 10000
F = 128
H = 128
K = 10
E = 320000

L = 16
NC = 2
NS = 16
NW = NC * NS

EPG = 64
N_PAD = 10240
NPT = N_PAD // NS
HPT = 640
HLAST = N - (NS - 1) * HPT
E_PAD = 327680
GT = E_PAD // EPG
GPT = GT // NW
G_SC0 = 288
G_SC1 = (GT - NS * G_SC0) // NS
DGPT = GT // NS
CH = 16


def _rsqrt_newton(d):
    ib = lax.bitcast_convert_type(d, jnp.int32)
    y = lax.bitcast_convert_type(jnp.int32(0x5F3759DF) - (ib >> 1), jnp.float32)
    for _ in range(3):
        y = y * (1.5 - 0.5 * d * y * y)
    return y


def _sc_kernel_body(row2, col2, ew2, xw, part_out, dis_out,
                    nvec_sh, hacc_sh,
                    rowv, colv, eww, gbuf0, gbuf1, sbuf0, sbuf1,
                    degv, disl,
                    gsem0, gsem1, ssem0, ssem1, dsem):
    cid = lax.axis_index("c")
    sid = lax.axis_index("s")
    nbase = sid * NPT
    hbase = sid * HPT

    z = jnp.zeros((L,), jnp.float32)

    def zdeg(i, c):
        degv[pl.ds(i * L, L)] = z
        return c
    lax.fori_loop(0, NPT // L, zdeg, 0)
    pltpu.sync_copy(degv, nvec_sh.at[pl.ds(nbase, NPT)])

    def zrow(i, c):
        for v in range(H // L):
            sbuf0[i, pl.ds(v * L, L)] = z
        return c
    lax.fori_loop(0, EPG, zrow, 0)

    @pl.when(sid < NS - 1)
    def _():
        zs = [pltpu.async_copy(sbuf0, hacc_sh.at[pl.ds(hbase + r * EPG, EPG)],
                               dsem)
              for r in range(HPT // EPG)]
        for d in zs:
            d.wait()

    @pl.when(sid == NS - 1)
    def _():
        zs = [pltpu.async_copy(sbuf0, hacc_sh.at[pl.ds(hbase + r * EPG, EPG)],
                               dsem)
              for r in range(HLAST // EPG)]
        zs.append(pltpu.async_copy(
            sbuf0.at[pl.ds(0, HLAST % EPG)],
            hacc_sh.at[pl.ds(hbase + (HLAST // EPG) * EPG, HLAST % EPG)],
            dsem))
        for d in zs:
            d.wait()

    plsc.subcore_barrier()

    dg0 = sid * DGPT
    scope_deg = jax.named_scope("deg_phase")
    scope_deg.__enter__()

    def deg_chunk(ci, c):
        gb = dg0 + ci * CH
        pltpu.sync_copy(col2.at[pl.ds(gb, CH)], colv)
        pltpu.sync_copy(ew2.at[pl.ds(gb, CH)], eww)
        ds_ = [pltpu.async_copy(eww.at[g], nvec_sh.at[colv.at[g]], dsem,
                                add=True)
               for g in range(CH)]
        for d in ds_:
            d.wait()
        return c
    lax.fori_loop(0, DGPT // CH, deg_chunk, 0)

    plsc.subcore_barrier()
    scope_deg.__exit__(None, None, None)

    pltpu.sync_copy(nvec_sh.at[pl.ds(nbase, NPT)], degv)

    def dis_step(i, c):
        d = degv[pl.ds(i * L, L)] + 1.0
        degv[pl.ds(i * L, L)] = _rsqrt_newton(d)
        return c
    lax.fori_loop(0, NPT // L, dis_step, 0)

    plsc.subcore_barrier()

    pltpu.sync_copy(degv, nvec_sh.at[pl.ds(nbase, NPT)])

    @pl.when(cid == 0)
    def _():
        pltpu.sync_copy(degv, dis_out.at[pl.ds(nbase, NPT)])

    plsc.subcore_barrier()

    pltpu.sync_copy(nvec_sh, disl)
    scope_msg = jax.named_scope("msg_phase")
    scope_msg.__enter__()

    g0 = jnp.where(cid == 0, sid * G_SC0, NS * G_SC0 + sid * G_SC1)
    nchunk = jnp.where(cid == 0, G_SC0 // CH, G_SC1 // CH)

    def process(g, i, gbuf, gsem, sbuf, ssem):
        pltpu.make_async_copy(xw.at[rowv.at[g]], gbuf, gsem).wait()

        @pl.when(i > 0)
        def _():
            pltpu.make_async_copy(sbuf, hacc_sh.at[colv.at[g]], ssem).wait()

        def scale(j8, cc):
            ri = rowv[g, pl.ds(j8 * L, L)]
            ev = eww[g, pl.ds(j8 * L, L)]
            nv = plsc.load_gather(disl, [ri]) * ev
            for jj in range(L):
                s = nv[jj]
                j = j8 * L + jj
                for v in range(H // L):
                    sbuf[j, pl.ds(v * L, L)] = gbuf[j, pl.ds(v * L, L)] * s
            return cc
        lax.fori_loop(0, EPG // L, scale, 0)

        pltpu.async_copy(sbuf, hacc_sh.at[colv.at[g]], ssem, add=True)
        gg = g + 2

        @pl.when(gg < CH)
        def _():
            pltpu.async_copy(xw.at[rowv.at[gg]], gbuf, gsem)

    def msg_chunk(ci, c):
        gb = g0 + ci * CH
        pltpu.sync_copy(row2.at[pl.ds(gb, CH)], rowv)
        pltpu.sync_copy(col2.at[pl.ds(gb, CH)], colv)
        pltpu.sync_copy(ew2.at[pl.ds(gb, CH)], eww)
        pltpu.async_copy(xw.at[rowv.at[0]], gbuf0, gsem0)
        pltpu.async_copy(xw.at[rowv.at[1]], gbuf1, gsem1)

        def pair(i, cc):
            process(2 * i, i, gbuf0, gsem0, sbuf0, ssem0)
            process(2 * i + 1, i, gbuf1, gsem1, sbuf1, ssem1)
            return cc
        lax.fori_loop(0, CH // 2, pair, 0)

        pltpu.make_async_copy(sbuf0, hacc_sh.at[colv.at[0]], ssem0).wait()
        pltpu.make_async_copy(sbuf1, hacc_sh.at[colv.at[1]], ssem1).wait()
        return c
    lax.fori_loop(0, nchunk, msg_chunk, 0)

    plsc.subcore_barrier()
    scope_msg.__exit__(None, None, None)

    @pl.when(sid < NS - 1)
    def _():
        pltpu.sync_copy(hacc_sh.at[pl.ds(hbase, HPT)],
                        part_out.at[cid, pl.ds(hbase, HPT)])

    @pl.when(sid == NS - 1)
    def _():
        pltpu.sync_copy(hacc_sh.at[pl.ds(hbase, HLAST)],
                        part_out.at[cid, pl.ds(hbase, HLAST)])


def _make_sc_kernel():
    mesh = plsc.VectorSubcoreMesh(core_axis_name="c", subcore_axis_name="s")
    return functools.partial(
        pl.kernel,
        mesh=mesh,
        compiler_params=pltpu.CompilerParams(needs_layout_passes=False),
        out_type=[
            jax.ShapeDtypeStruct((NC, N, H), jnp.float32),
            jax.ShapeDtypeStruct((N_PAD,), jnp.float32),
        ],
        scratch_types=[
            pltpu.VMEM_SHARED((N_PAD,), jnp.float32),
            pltpu.VMEM_SHARED((N, H), jnp.float32),
            pltpu.VMEM((CH, EPG), jnp.int32),
            pltpu.VMEM((CH, EPG), jnp.int32),
            pltpu.VMEM((CH, EPG), jnp.float32),
            pltpu.VMEM((EPG, H), jnp.float32),
            pltpu.VMEM((EPG, H), jnp.float32),
            pltpu.VMEM((EPG, H), jnp.float32),
            pltpu.VMEM((EPG, H), jnp.float32),
            pltpu.VMEM((NPT,), jnp.float32),
            pltpu.VMEM((N_PAD,), jnp.float32),
            pltpu.SemaphoreType.DMA,
            pltpu.SemaphoreType.DMA,
            pltpu.SemaphoreType.DMA,
            pltpu.SemaphoreType.DMA,
            pltpu.SemaphoreType.DMA,
        ],
    )(_sc_kernel_body)


def _mm_body(x_ref, w_ref, o_ref):
    o_ref[...] = jnp.dot(x_ref[...], w_ref[...],
                         preferred_element_type=jnp.float32)


def _fin_body(p0, p1, xwr, disr, br, mur, hout, qout):
    d = disr[...]
    h = d * (p0[...] + p1[...]) + (d * d) * xwr[...] + br[...]
    hout[...] = h
    cols = []
    for k in range(K):
        mk = mur[k, :][None, :]
        dd = h - mk
        cols.append(jnp.sum(dd * dd, axis=1, keepdims=True))
    d2 = jnp.concatenate(cols, axis=1)
    u = 1.0 / (1.0 + d2 / ALPHA + 1e-8)
    qq = jnp.exp((ALPHA + 1.0) * jnp.log(u))
    qout[...] = qq / jnp.sum(qq, axis=1, keepdims=True)


def kernel(x, edge_index, edge_attr, W, b, mu):
    row = edge_index[0]
    col = edge_index[1]
    pad = E_PAD - E
    row2 = jnp.concatenate([row, jnp.zeros((pad,), jnp.int32)]).reshape(GT, EPG)
    col2 = jnp.concatenate([col, jnp.zeros((pad,), jnp.int32)]).reshape(GT, EPG)
    ew2 = jnp.concatenate([edge_attr,
                           jnp.zeros((pad,), jnp.float32)]).reshape(GT, EPG)

    BLK = 1000
    xw = pl.pallas_call(
        _mm_body,
        grid=(N // BLK,),
        in_specs=[pl.BlockSpec((BLK, F), lambda i: (i, 0)),
                  pl.BlockSpec((F, H), lambda i: (0, 0))],
        out_specs=pl.BlockSpec((BLK, H), lambda i: (i, 0)),
        out_shape=jax.ShapeDtypeStruct((N, H), jnp.float32),
    )(x, W)

    parts, dis = _make_sc_kernel()(row2, col2, ew2, xw)

    p0 = parts[0, :N]
    p1 = parts[1, :N]
    dis2 = dis[:N].reshape(N, 1)
    b2 = b.reshape(1, H)

    h, q = pl.pallas_call(
        _fin_body,
        grid=(N // BLK,),
        in_specs=[pl.BlockSpec((BLK, H), lambda i: (i, 0)),
                  pl.BlockSpec((BLK, H), lambda i: (i, 0)),
                  pl.BlockSpec((BLK, H), lambda i: (i, 0)),
                  pl.BlockSpec((BLK, 1), lambda i: (i, 0)),
                  pl.BlockSpec((1, H), lambda i: (0, 0)),
                  pl.BlockSpec((K, H), lambda i: (0, 0))],
        out_specs=[pl.BlockSpec((BLK, H), lambda i: (i, 0)),
                   pl.BlockSpec((BLK, K), lambda i: (i, 0))],
        out_shape=[jax.ShapeDtypeStruct((N, H), jnp.float32),
                   jax.ShapeDtypeStruct((N, K), jnp.float32)],
    )(p0, p1, xw, dis2, b2, mu)

    return (h, q)
